# baseline (speedup 1.0000x reference)
import jax
import jax.numpy as jnp
from jax.experimental import pallas as pl
from jax.experimental.pallas import tpu as pltpu

_BATCH = 16
_N = 32
_S = 2048
_L = 2048
_D = 768
_C = 256
_NCH = _L // _C


def _kern(se_ref, att_ref, ctx_hbm, out_ref, scr_ref, sem):
    b = pl.program_id(0)
    att = att_ref[0, :, :]
    starts = se_ref[0, :, 0].reshape(_N, 1)
    ends = se_ref[0, :, 1].reshape(_N, 1)
    pos = jax.lax.broadcasted_iota(jnp.int32, (_N, _S), 1)
    m = (pos >= starts) & (pos < ends)
    sums = jnp.sum(jnp.where(m, att, 0.0), axis=1, keepdims=True)
    maxv = jnp.max(sums)
    idx = jax.lax.broadcasted_iota(jnp.int32, (_N, 1), 0)
    best = jnp.min(jnp.where(sums == maxv, idx, _N))
    sel = maxv > 0.0
    is_best = idx == best
    start = jnp.where(sel, jnp.sum(jnp.where(is_best, starts, 0)), 0)
    end = jnp.where(sel, jnp.sum(jnp.where(is_best, ends, 0)), 0)
    nv = end - start

    base = b * _S
    _W = _C + 8

    def _woff(lo):
        roff = jnp.minimum((start + lo) // 8 * 8, _S - _W)
        return pl.multiple_of(roff, 8)

    for c in range(_NCH):
        lo = c * _C

        @pl.when(nv > lo)
        def _():
            pltpu.make_async_copy(
                ctx_hbm.at[pl.ds(base + _woff(lo), _W), :],
                scr_ref.at[c], sem,
            ).start()

    for c in range(_NCH):
        lo = c * _C

        @pl.when(nv <= lo)
        def _():
            out_ref[0, lo:lo + _C, :] = jnp.zeros((_C, _D), jnp.float32)

    for c in range(_NCH):
        lo = c * _C

        @pl.when(nv > lo)
        def _():
            roff = _woff(lo)
            delta = start + lo - roff
            pltpu.make_async_copy(
                ctx_hbm.at[pl.ds(base + roff, _W), :],
                scr_ref.at[c], sem,
            ).wait()
            win = scr_ref[c]
            shift = jax.lax.rem(_W - delta, _W)
            rot = pltpu.roll(win, shift, axis=0)
            rows = jax.lax.broadcasted_iota(jnp.int32, (_C, 1), 0)
            out_ref[0, lo:lo + _C, :] = jnp.where(
                rows < (nv - lo), rot[0:_C, :], 0.0)


@jax.jit
def kernel(startends, attention, context):
    att3 = attention.reshape(_BATCH, 1, _S)
    ctx_flat = context.reshape(_BATCH * _S, _D)
    return pl.pallas_call(
        _kern,
        grid=(_BATCH,),
        in_specs=[
            pl.BlockSpec((1, _N, 2), lambda b: (b, 0, 0)),
            pl.BlockSpec((1, 1, _S), lambda b: (b, 0, 0)),
            pl.BlockSpec(memory_space=pltpu.MemorySpace.HBM),
        ],
        out_specs=pl.BlockSpec((1, _L, _D), lambda b: (b, 0, 0)),
        out_shape=jax.ShapeDtypeStruct((_BATCH, _L, _D), jnp.float32),
        scratch_shapes=[
            pltpu.VMEM((_NCH, _C + 8, _D), jnp.float32),
            pltpu.SemaphoreType.DMA,
        ],
        compiler_params=pltpu.CompilerParams(
            dimension_semantics=("arbitrary",)),
    )(startends, att3, ctx_flat)


# speedup vs baseline: 1.3020x; 1.3020x over previous
import jax
import jax.numpy as jnp
from jax.experimental import pallas as pl
from jax.experimental.pallas import tpu as pltpu

_BATCH = 16
_N = 32
_S = 2048
_L = 2048
_D = 768
_C = 256
_NCH = _L // _C


def _kern(se_ref, att_ref, ctx_ref, out_ref):
    att = att_ref[0, :, :]
    starts = se_ref[0, :, 0].reshape(_N, 1)
    ends = se_ref[0, :, 1].reshape(_N, 1)
    pos = jax.lax.broadcasted_iota(jnp.int32, (_N, _S), 1)
    m = (pos >= starts) & (pos < ends)
    sums = jnp.sum(jnp.where(m, att, 0.0), axis=1, keepdims=True)
    maxv = jnp.max(sums)
    idx = jax.lax.broadcasted_iota(jnp.int32, (_N, 1), 0)
    best = jnp.min(jnp.where(sums == maxv, idx, _N))
    sel = maxv > 0.0
    is_best = idx == best
    start = jnp.where(sel, jnp.sum(jnp.where(is_best, starts, 0)), 0)
    end = jnp.where(sel, jnp.sum(jnp.where(is_best, ends, 0)), 0)
    nv = end - start

    _W = _C + 8
    for c in range(_NCH):
        lo = c * _C

        @pl.when(nv <= lo)
        def _():
            out_ref[0, lo:lo + _C, :] = jnp.zeros((_C, _D), jnp.float32)

        @pl.when(nv > lo)
        def _():
            roff = jnp.minimum((start + lo) // 8 * 8, _S - _W)
            roff = pl.multiple_of(roff, 8)
            t = start + lo - roff
            win = ctx_ref[0, pl.ds(roff, _W), :]
            shift = jax.lax.rem(_W - t, _W)
            rot = pltpu.roll(win, shift, axis=0)
            rows = jax.lax.broadcasted_iota(jnp.int32, (_C, 1), 0)
            valid = rows < (nv - lo)
            out_ref[0, lo:lo + _C, :] = jnp.where(
                valid, rot[0:_C, :], 0.0)


@jax.jit
def kernel(startends, attention, context):
    att3 = attention.reshape(_BATCH, 1, _S)
    return pl.pallas_call(
        _kern,
        grid=(_BATCH,),
        in_specs=[
            pl.BlockSpec((1, _N, 2), lambda b: (b, 0, 0)),
            pl.BlockSpec((1, 1, _S), lambda b: (b, 0, 0)),
            pl.BlockSpec((1, _S, _D), lambda b: (b, 0, 0)),
        ],
        out_specs=pl.BlockSpec((1, _L, _D), lambda b: (b, 0, 0)),
        out_shape=jax.ShapeDtypeStruct((_BATCH, _L, _D), jnp.float32),
        compiler_params=pltpu.CompilerParams(
            dimension_semantics=("arbitrary",)),
    )(startends, att3, context)


# speedup vs baseline: 1.4020x; 1.0768x over previous
import jax
import jax.numpy as jnp
from jax.experimental import pallas as pl
from jax.experimental.pallas import tpu as pltpu

_BATCH = 16
_N = 32
_S = 2048
_L = 2048
_D = 768
_C = 256
_NCH = _L // _C


def _kern(se_ref, att_ref, ctx_ref, out_ref):
    att = att_ref[0, :, :]
    starts = se_ref[0, :, 0].reshape(_N, 1)
    ends = se_ref[0, :, 1].reshape(_N, 1)
    pos = jax.lax.broadcasted_iota(jnp.int32, (_N, _S), 1)
    m = (pos >= starts) & (pos < ends)
    sums = jnp.sum(jnp.where(m, att, 0.0), axis=1, keepdims=True)
    maxv = jnp.max(sums)
    idx = jax.lax.broadcasted_iota(jnp.int32, (_N, 1), 0)
    best = jnp.min(jnp.where(sums == maxv, idx, _N))
    sel = maxv > 0.0
    is_best = idx == best
    start = jnp.where(sel, jnp.sum(jnp.where(is_best, starts, 0)), 0)
    end = jnp.where(sel, jnp.sum(jnp.where(is_best, ends, 0)), 0)
    nv = end - start

    _W = _C + 8
    _G = _C // 8
    d8 = jax.lax.rem(start, 8)
    for c in range(_NCH):
        lo = c * _C
        roff_raw = (start + lo) // 8 * 8
        clamped = roff_raw > _S - _W

        @pl.when(nv <= lo)
        def _():
            out_ref[0, lo:lo + _C, :] = jnp.zeros((_C, _D), jnp.float32)

        def _fast(masked):
            roff = pl.multiple_of(jnp.minimum(roff_raw, _S - _W), 8)
            win = ctx_ref[0, pl.ds(roff, _W), :]
            w3 = win.reshape(_W // 8, 8, _D)
            rolled = pltpu.roll(w3, jax.lax.rem(8 - d8, 8), axis=1)
            sub = jax.lax.broadcasted_iota(jnp.int32, (_G, 8, 1), 1)
            rot3 = jnp.where(sub < 8 - d8,
                             rolled[0:_G, :, :], rolled[1:_G + 1, :, :])
            if masked:
                grp = jax.lax.broadcasted_iota(jnp.int32, (_G, 8, 1), 0)
                rot3 = jnp.where(grp * 8 + sub < (nv - lo), rot3, 0.0)
            out_ref[0, lo:lo + _C, :] = rot3.reshape(_C, _D)

        def _slow():
            roff = pl.multiple_of(jnp.minimum(roff_raw, _S - _W), 8)
            t = start + lo - roff
            win = ctx_ref[0, pl.ds(roff, _W), :]
            shift = jax.lax.rem(_W - t, _W)
            rot = pltpu.roll(win, shift, axis=0)
            rows = jax.lax.broadcasted_iota(jnp.int32, (_C, 1), 0)
            out_ref[0, lo:lo + _C, :] = jnp.where(
                rows < (nv - lo), rot[0:_C, :], 0.0)

        full = nv >= lo + _C
        tail = (nv > lo) & (nv < lo + _C)
        pl.when(full & jnp.logical_not(clamped))(lambda: _fast(False))
        pl.when(tail & jnp.logical_not(clamped))(lambda: _fast(True))
        pl.when((nv > lo) & clamped)(_slow)


@jax.jit
def kernel(startends, attention, context):
    att3 = attention.reshape(_BATCH, 1, _S)
    return pl.pallas_call(
        _kern,
        grid=(_BATCH,),
        in_specs=[
            pl.BlockSpec((1, _N, 2), lambda b: (b, 0, 0)),
            pl.BlockSpec((1, 1, _S), lambda b: (b, 0, 0)),
            pl.BlockSpec((1, _S, _D), lambda b: (b, 0, 0)),
        ],
        out_specs=pl.BlockSpec((1, _L, _D), lambda b: (b, 0, 0)),
        out_shape=jax.ShapeDtypeStruct((_BATCH, _L, _D), jnp.float32),
        compiler_params=pltpu.CompilerParams(
            dimension_semantics=("arbitrary",)),
    )(startends, att3, context)
